# baseline (speedup 1.0000x reference)
import functools

import jax
import jax.numpy as jnp
from jax.experimental import pallas as pl
from jax.experimental.pallas import tpu as pltpu

EPS = 1e-6
H = 4096
M_TILE = 256
N_SLICE = 1024


def _layer_kernel(*refs, relu_in, final_norm, cast_next, emit_ssq, take_ssq):
    it = iter(refs)
    x_ref = next(it)
    w_ref = next(it)
    gin_ref = next(it)
    gout_ref = next(it)
    ssqin_ref = next(it) if take_ssq else None
    wn_ref = next(it) if cast_next else None
    o_ref = next(it)
    ssqout_ref = next(it) if emit_ssq else None
    wc_ref = next(it) if cast_next else None

    track_ssq = emit_ssq or final_norm
    xb = x_ref[...].astype(jnp.float32)
    if relu_in:
        xb = jnp.maximum(xb, 0.0)
    if take_ssq:
        var = ssqin_ref[...] * (1.0 / H)
    else:
        var = jnp.mean(xb * xb, axis=1, keepdims=True)
    y = (xb * jax.lax.rsqrt(var + EPS) * gin_ref[...]).astype(jnp.bfloat16)
    ssq = None
    for n in range(0, H, N_SLICE):
        sl = slice(n, n + N_SLICE)
        acc = jnp.dot(y, w_ref[:, sl], preferred_element_type=jnp.float32)
        o = acc + xb[:, sl]
        o_ref[:, sl] = o.astype(o_ref.dtype)
        if track_ssq:
            p = jnp.sum(o * o, axis=1, keepdims=True)
            ssq = p if ssq is None else ssq + p
    if emit_ssq:
        ssqout_ref[...] = ssq
    if final_norm:
        rstd2 = jax.lax.rsqrt(ssq * (1.0 / H) + EPS)
        o_ref[...] = o_ref[...] * rstd2 * gout_ref[...]
    if cast_next:
        wc_ref[...] = wn_ref[...].astype(jnp.bfloat16)


def _layer(x, w_bf16, g_in, g_out, w_next=None, ssq_in=None, *, relu_in,
           final_norm, emit_ssq, m_tile=M_TILE, out_dtype=jnp.float32,
           interpret=False):
    t = x.shape[0]
    steps = t // m_tile
    cast_next = w_next is not None
    take_ssq = ssq_in is not None
    body = functools.partial(_layer_kernel, relu_in=relu_in,
                             final_norm=final_norm, cast_next=cast_next,
                             emit_ssq=emit_ssq, take_ssq=take_ssq)
    in_specs = [
        pl.BlockSpec((m_tile, H), lambda i: (i, 0)),
        pl.BlockSpec((H, H), lambda i: (0, 0)),
        pl.BlockSpec((1, H), lambda i: (0, 0)),
        pl.BlockSpec((1, H), lambda i: (0, 0)),
    ]
    args = [x, w_bf16, g_in, g_out]
    if take_ssq:
        in_specs.append(pl.BlockSpec((m_tile, 1), lambda i: (i, 0)))
        args.append(ssq_in)
    out_shape = [jax.ShapeDtypeStruct((t, H), out_dtype)]
    out_specs = [pl.BlockSpec((m_tile, H), lambda i: (i, 0))]
    if emit_ssq:
        out_shape.append(jax.ShapeDtypeStruct((t, 1), jnp.float32))
        out_specs.append(pl.BlockSpec((m_tile, 1), lambda i: (i, 0)))
    if cast_next:
        wrows = H // steps
        in_specs.append(pl.BlockSpec((wrows, H), lambda i: (i, 0)))
        args.append(w_next)
        out_shape.append(jax.ShapeDtypeStruct((H, H), jnp.bfloat16))
        out_specs.append(pl.BlockSpec((wrows, H), lambda i: (i, 0)))
    return pl.pallas_call(
        body,
        out_shape=out_shape,
        grid=(steps,),
        in_specs=in_specs,
        out_specs=out_specs,
        compiler_params=pltpu.CompilerParams(
            dimension_semantics=("parallel",),
            vmem_limit_bytes=61 * 1024 * 1024,
        ),
        name=f"rmsnorm_mm_{'relu' if relu_in else 'fin' if final_norm else 'mid'}",
        interpret=interpret,
    )(*args)


def kernel(x, w0, w1, w2, g0, g1, g2, g3, *, interpret=False):
    w0b = w0.astype(jnp.bfloat16)
    g0r = g0.reshape(1, H)
    g1r = g1.reshape(1, H)
    g2r = g2.reshape(1, H)
    g3r = g3.reshape(1, H)
    r1, ssq1, w1b = _layer(x, w0b, g0r, g1r, w1, relu_in=True,
                           final_norm=False, emit_ssq=True,
                           out_dtype=jnp.bfloat16, interpret=interpret)
    r2, ssq2, w2b = _layer(r1, w1b, g1r, g2r, w2, ssq1, relu_in=False,
                           final_norm=False, emit_ssq=True,
                           out_dtype=jnp.bfloat16, interpret=interpret)
    (y4,) = _layer(r2, w2b, g2r, g3r, None, ssq2, relu_in=False,
                   final_norm=True, emit_ssq=False,
                   interpret=interpret)
    return y4


# speedup vs baseline: 1.0008x; 1.0008x over previous
import functools

import jax
import jax.numpy as jnp
from jax.experimental import pallas as pl
from jax.experimental.pallas import tpu as pltpu

EPS = 1e-6
H = 4096
M_TILE = 256
N_SLICE = 1024


def _layer_kernel(*refs, relu_in, final_norm, cast_next, emit_ssq, take_ssq):
    it = iter(refs)
    x_ref = next(it)
    w_ref = next(it)
    gin_ref = next(it)
    gout_ref = next(it)
    ssqin_ref = next(it) if take_ssq else None
    wn_ref = next(it) if cast_next else None
    o_ref = next(it)
    ssqout_ref = next(it) if emit_ssq else None
    wc_ref = next(it) if cast_next else None

    track_ssq = emit_ssq or final_norm
    xb = x_ref[...].astype(jnp.float32)
    if relu_in:
        xb = jnp.maximum(xb, 0.0)
    if take_ssq:
        var = ssqin_ref[...] * (1.0 / H)
    else:
        var = jnp.mean(xb * xb, axis=1, keepdims=True)
    y = (xb * jax.lax.rsqrt(var + EPS) * gin_ref[...]).astype(jnp.bfloat16)
    ssq = None
    for n in range(0, H, N_SLICE):
        sl = slice(n, n + N_SLICE)
        acc = jnp.dot(y, w_ref[:, sl], preferred_element_type=jnp.float32)
        o = acc + xb[:, sl]
        o_ref[:, sl] = o.astype(o_ref.dtype)
        if track_ssq:
            p = jnp.sum(o * o, axis=1, keepdims=True)
            ssq = p if ssq is None else ssq + p
    if emit_ssq:
        ssqout_ref[...] = ssq
    if final_norm:
        rstd2 = jax.lax.rsqrt(ssq * (1.0 / H) + EPS)
        o_ref[...] = o_ref[...] * rstd2 * gout_ref[...]
    if cast_next:
        wc_ref[...] = wn_ref[...].astype(jnp.bfloat16)


def _layer(x, w_bf16, g_in, g_out, w_next=None, ssq_in=None, *, relu_in,
           final_norm, emit_ssq, m_tile=M_TILE, out_dtype=jnp.float32,
           interpret=False):
    t = x.shape[0]
    steps = t // m_tile
    cast_next = w_next is not None
    take_ssq = ssq_in is not None
    body = functools.partial(_layer_kernel, relu_in=relu_in,
                             final_norm=final_norm, cast_next=cast_next,
                             emit_ssq=emit_ssq, take_ssq=take_ssq)
    in_specs = [
        pl.BlockSpec((m_tile, H), lambda i: (i, 0)),
        pl.BlockSpec((H, H), lambda i: (0, 0)),
        pl.BlockSpec((1, H), lambda i: (0, 0)),
        pl.BlockSpec((1, H), lambda i: (0, 0)),
    ]
    args = [x, w_bf16, g_in, g_out]
    if take_ssq:
        in_specs.append(pl.BlockSpec((m_tile, 1), lambda i: (i, 0)))
        args.append(ssq_in)
    out_shape = [jax.ShapeDtypeStruct((t, H), out_dtype)]
    out_specs = [pl.BlockSpec((m_tile, H), lambda i: (i, 0))]
    if emit_ssq:
        out_shape.append(jax.ShapeDtypeStruct((t, 1), jnp.float32))
        out_specs.append(pl.BlockSpec((m_tile, 1), lambda i: (i, 0)))
    if cast_next:
        wrows = H // steps
        in_specs.append(pl.BlockSpec((wrows, H), lambda i: (i, 0)))
        args.append(w_next)
        out_shape.append(jax.ShapeDtypeStruct((H, H), jnp.bfloat16))
        out_specs.append(pl.BlockSpec((wrows, H), lambda i: (i, 0)))
    return pl.pallas_call(
        body,
        out_shape=out_shape,
        grid=(steps,),
        in_specs=in_specs,
        out_specs=out_specs,
        compiler_params=pltpu.CompilerParams(
            dimension_semantics=("parallel",),
            vmem_limit_bytes=61 * 1024 * 1024,
        ),
        name=f"rmsnorm_mm_{'relu' if relu_in else 'fin' if final_norm else 'mid'}",
        interpret=interpret,
    )(*args)


def kernel(x, w0, w1, w2, g0, g1, g2, g3, *, interpret=False):
    w0b = w0.astype(jnp.bfloat16)
    g0r = g0.reshape(1, H)
    g1r = g1.reshape(1, H)
    g2r = g2.reshape(1, H)
    g3r = g3.reshape(1, H)
    r1, ssq1, w1b = _layer(x, w0b, g0r, g1r, w1, relu_in=True,
                           final_norm=False, emit_ssq=True,
                           interpret=interpret)
    r2, ssq2, w2b = _layer(r1, w1b, g1r, g2r, w2, ssq1, relu_in=False,
                           final_norm=False, emit_ssq=True,
                           interpret=interpret)
    (y4,) = _layer(r2, w2b, g2r, g3r, None, ssq2, relu_in=False,
                   final_norm=True, emit_ssq=False,
                   interpret=interpret)
    return y4


# speedup vs baseline: 1.0036x; 1.0027x over previous
import functools

import jax
import jax.numpy as jnp
from jax.experimental import pallas as pl
from jax.experimental.pallas import tpu as pltpu

EPS = 1e-6
H = 4096
M_TILE = 256
N_SLICE = 1024


def _layer_kernel(*refs, relu_in, final_norm, cast_next, emit_ssq, take_ssq):
    it = iter(refs)
    x_ref = next(it)
    w_ref = next(it)
    gin_ref = next(it)
    gout_ref = next(it)
    ssqin_ref = next(it) if take_ssq else None
    wn_ref = next(it) if cast_next else None
    o_ref = next(it)
    ssqout_ref = next(it) if emit_ssq else None
    wc_ref = next(it) if cast_next else None

    track_ssq = emit_ssq or final_norm
    xb = x_ref[...].astype(jnp.float32)
    if relu_in:
        xb = jnp.maximum(xb, 0.0)
    if take_ssq:
        var = ssqin_ref[...] * (1.0 / H)
    else:
        ssq0 = None
        for c in range(0, H, 512):
            xc = xb[:, c:c + 512]
            p0 = jnp.sum(xc * xc, axis=1, keepdims=True)
            ssq0 = p0 if ssq0 is None else ssq0 + p0
        var = ssq0 * (1.0 / H)
    y = (xb * jax.lax.rsqrt(var + EPS) * gin_ref[...]).astype(jnp.bfloat16)
    ssq = None
    n_slices = H // N_SLICE
    crows = wn_ref.shape[0] // n_slices if cast_next else 0
    for idx, n in enumerate(range(0, H, N_SLICE)):
        sl = slice(n, n + N_SLICE)
        acc = jnp.dot(y, w_ref[:, sl], preferred_element_type=jnp.float32)
        o = acc + xb[:, sl]
        o_ref[:, sl] = o.astype(o_ref.dtype)
        if track_ssq:
            p = jnp.sum(o * o, axis=1, keepdims=True)
            ssq = p if ssq is None else ssq + p
        if cast_next:
            cs = slice(idx * crows, (idx + 1) * crows)
            wc_ref[cs, :] = wn_ref[cs, :].astype(jnp.bfloat16)
    if emit_ssq:
        ssqout_ref[...] = ssq
    if final_norm:
        rstd2 = jax.lax.rsqrt(ssq * (1.0 / H) + EPS)
        o_ref[...] = o_ref[...] * rstd2 * gout_ref[...]


def _layer(x, w_bf16, g_in, g_out, w_next=None, ssq_in=None, *, relu_in,
           final_norm, emit_ssq, m_tile=M_TILE, out_dtype=jnp.float32,
           interpret=False):
    t = x.shape[0]
    steps = t // m_tile
    cast_next = w_next is not None
    take_ssq = ssq_in is not None
    body = functools.partial(_layer_kernel, relu_in=relu_in,
                             final_norm=final_norm, cast_next=cast_next,
                             emit_ssq=emit_ssq, take_ssq=take_ssq)
    in_specs = [
        pl.BlockSpec((m_tile, H), lambda i: (i, 0)),
        pl.BlockSpec((H, H), lambda i: (0, 0)),
        pl.BlockSpec((1, H), lambda i: (0, 0)),
        pl.BlockSpec((1, H), lambda i: (0, 0)),
    ]
    args = [x, w_bf16, g_in, g_out]
    if take_ssq:
        in_specs.append(pl.BlockSpec((m_tile, 1), lambda i: (i, 0)))
        args.append(ssq_in)
    out_shape = [jax.ShapeDtypeStruct((t, H), out_dtype)]
    out_specs = [pl.BlockSpec((m_tile, H), lambda i: (i, 0))]
    if emit_ssq:
        out_shape.append(jax.ShapeDtypeStruct((t, 1), jnp.float32))
        out_specs.append(pl.BlockSpec((m_tile, 1), lambda i: (i, 0)))
    if cast_next:
        wrows = H // steps
        in_specs.append(pl.BlockSpec((wrows, H), lambda i: (i, 0)))
        args.append(w_next)
        out_shape.append(jax.ShapeDtypeStruct((H, H), jnp.bfloat16))
        out_specs.append(pl.BlockSpec((wrows, H), lambda i: (i, 0)))
    return pl.pallas_call(
        body,
        out_shape=out_shape,
        grid=(steps,),
        in_specs=in_specs,
        out_specs=out_specs,
        compiler_params=pltpu.CompilerParams(
            dimension_semantics=("parallel",),
            vmem_limit_bytes=61 * 1024 * 1024,
        ),
        name=f"rmsnorm_mm_{'relu' if relu_in else 'fin' if final_norm else 'mid'}",
        interpret=interpret,
    )(*args)


def kernel(x, w0, w1, w2, g0, g1, g2, g3, *, interpret=False):
    w0b = w0.astype(jnp.bfloat16)
    g0r = g0.reshape(1, H)
    g1r = g1.reshape(1, H)
    g2r = g2.reshape(1, H)
    g3r = g3.reshape(1, H)
    r1, ssq1, w1b = _layer(x, w0b, g0r, g1r, w1, relu_in=True,
                           final_norm=False, emit_ssq=True,
                           interpret=interpret)
    r2, ssq2, w2b = _layer(r1, w1b, g1r, g2r, w2, ssq1, relu_in=False,
                           final_norm=False, emit_ssq=True,
                           interpret=interpret)
    (y4,) = _layer(r2, w2b, g2r, g3r, None, ssq2, relu_in=False,
                   final_norm=True, emit_ssq=False,
                   interpret=interpret)
    return y4


# speedup vs baseline: 1.0054x; 1.0018x over previous
import functools

import jax
import jax.numpy as jnp
from jax.experimental import pallas as pl
from jax.experimental.pallas import tpu as pltpu

EPS = 1e-6
H = 4096
M_TILE = 256
N_SLICE = 1024


def _layer_kernel(*refs, relu_in, final_norm, cast_next, emit_ssq, take_ssq):
    it = iter(refs)
    x_ref = next(it)
    w_ref = next(it)
    gin_ref = next(it)
    gout_ref = next(it)
    ssqin_ref = next(it) if take_ssq else None
    wn_ref = next(it) if cast_next else None
    o_ref = next(it)
    ssqout_ref = next(it) if emit_ssq else None
    wc_ref = next(it) if cast_next else None

    track_ssq = emit_ssq or final_norm
    xb = x_ref[...].astype(jnp.float32)
    if relu_in:
        xb = jnp.maximum(xb, 0.0)
    if take_ssq:
        var = ssqin_ref[...] * (1.0 / H)
    else:
        ssq0 = None
        for c in range(0, H, 512):
            xc = xb[:, c:c + 512]
            p0 = jnp.sum(xc * xc, axis=1, keepdims=True)
            ssq0 = p0 if ssq0 is None else ssq0 + p0
        var = ssq0 * (1.0 / H)
    y = (xb * jax.lax.rsqrt(var + EPS) * gin_ref[...]).astype(jnp.bfloat16)
    ssq = None
    n_slices = H // N_SLICE
    crows = wn_ref.shape[0] // n_slices if cast_next else 0
    o_parts = []
    for idx, n in enumerate(range(0, H, N_SLICE)):
        sl = slice(n, n + N_SLICE)
        acc = jnp.dot(y, w_ref[:, sl], preferred_element_type=jnp.float32)
        o = acc + xb[:, sl]
        if final_norm:
            o_parts.append(o)
        else:
            o_ref[:, sl] = o.astype(o_ref.dtype)
        if track_ssq:
            p = jnp.sum(o * o, axis=1, keepdims=True)
            ssq = p if ssq is None else ssq + p
        if cast_next:
            cs = slice(idx * crows, (idx + 1) * crows)
            wc_ref[cs, :] = wn_ref[cs, :].astype(jnp.bfloat16)
    if emit_ssq:
        ssqout_ref[...] = ssq
    if final_norm:
        rstd2 = jax.lax.rsqrt(ssq * (1.0 / H) + EPS)
        for idx, n in enumerate(range(0, H, N_SLICE)):
            sl = slice(n, n + N_SLICE)
            o_ref[:, sl] = o_parts[idx] * rstd2 * gout_ref[:, sl]


def _layer(x, w_bf16, g_in, g_out, w_next=None, ssq_in=None, *, relu_in,
           final_norm, emit_ssq, m_tile=M_TILE, out_dtype=jnp.float32,
           interpret=False):
    t = x.shape[0]
    steps = t // m_tile
    cast_next = w_next is not None
    take_ssq = ssq_in is not None
    body = functools.partial(_layer_kernel, relu_in=relu_in,
                             final_norm=final_norm, cast_next=cast_next,
                             emit_ssq=emit_ssq, take_ssq=take_ssq)
    in_specs = [
        pl.BlockSpec((m_tile, H), lambda i: (i, 0)),
        pl.BlockSpec((H, H), lambda i: (0, 0)),
        pl.BlockSpec((1, H), lambda i: (0, 0)),
        pl.BlockSpec((1, H), lambda i: (0, 0)),
    ]
    args = [x, w_bf16, g_in, g_out]
    if take_ssq:
        in_specs.append(pl.BlockSpec((m_tile, 1), lambda i: (i, 0)))
        args.append(ssq_in)
    out_shape = [jax.ShapeDtypeStruct((t, H), out_dtype)]
    out_specs = [pl.BlockSpec((m_tile, H), lambda i: (i, 0))]
    if emit_ssq:
        out_shape.append(jax.ShapeDtypeStruct((t, 1), jnp.float32))
        out_specs.append(pl.BlockSpec((m_tile, 1), lambda i: (i, 0)))
    if cast_next:
        wrows = H // steps
        in_specs.append(pl.BlockSpec((wrows, H), lambda i: (i, 0)))
        args.append(w_next)
        out_shape.append(jax.ShapeDtypeStruct((H, H), jnp.bfloat16))
        out_specs.append(pl.BlockSpec((wrows, H), lambda i: (i, 0)))
    return pl.pallas_call(
        body,
        out_shape=out_shape,
        grid=(steps,),
        in_specs=in_specs,
        out_specs=out_specs,
        compiler_params=pltpu.CompilerParams(
            dimension_semantics=("parallel",),
            vmem_limit_bytes=61 * 1024 * 1024,
        ),
        name=f"rmsnorm_mm_{'relu' if relu_in else 'fin' if final_norm else 'mid'}",
        interpret=interpret,
    )(*args)


def kernel(x, w0, w1, w2, g0, g1, g2, g3, *, interpret=False):
    w0b = w0.astype(jnp.bfloat16)
    g0r = g0.reshape(1, H)
    g1r = g1.reshape(1, H)
    g2r = g2.reshape(1, H)
    g3r = g3.reshape(1, H)
    r1, ssq1, w1b = _layer(x, w0b, g0r, g1r, w1, relu_in=True,
                           final_norm=False, emit_ssq=True,
                           interpret=interpret)
    r2, ssq2, w2b = _layer(r1, w1b, g1r, g2r, w2, ssq1, relu_in=False,
                           final_norm=False, emit_ssq=True,
                           interpret=interpret)
    (y4,) = _layer(r2, w2b, g2r, g3r, None, ssq2, relu_in=False,
                   final_norm=True, emit_ssq=False,
                   interpret=interpret)
    return y4
